# baseline (device time: 56726 ns/iter reference)
import jax
import jax.numpy as jnp
from jax import lax
from jax.experimental import pallas as pl
from jax.experimental.pallas import tpu as pltpu

N_DEV = 4
B, SQ, H, D = 4, 32, 8, 128
BH = B * H
H2 = H // 2
SCALE = D ** -0.5
NSLOTS = 6


def _stat_col(b, h):
    return 2 * (h * B + b)


def _flash_partial_body(q_ref, k_hbm, v_hbm, o_ref, st_ref,
                        kbuf, vbuf, ksems, vsems):
    def start_fetch(idx, slot):
        b, h = idx // H, idx % H
        pltpu.make_async_copy(
            k_hbm.at[b, :, h, :], kbuf.at[slot], ksems.at[slot]
        ).start()
        pltpu.make_async_copy(
            v_hbm.at[b, :, h, :], vbuf.at[slot], vsems.at[slot]
        ).start()

    def wait_fetch(idx, slot):
        b, h = idx // H, idx % H
        pltpu.make_async_copy(
            k_hbm.at[b, :, h, :], kbuf.at[slot], ksems.at[slot]
        ).wait()
        pltpu.make_async_copy(
            v_hbm.at[b, :, h, :], vbuf.at[slot], vsems.at[slot]
        ).wait()

    for j in range(NSLOTS - 1):
        start_fetch(j, j)
    for idx in range(BH):
        b, h = idx // H, idx % H
        slot = idx % NSLOTS
        nxt = idx + NSLOTS - 1
        if nxt < BH:
            start_fetch(nxt, nxt % NSLOTS)
        wait_fetch(idx, slot)

        q = q_ref[b, :, h, :] * SCALE
        s = lax.dot_general(
            q, kbuf[slot], (((1,), (1,)), ((), ())),
            preferred_element_type=jnp.float32,
        )
        m = jnp.max(s, axis=1, keepdims=True)
        p = jnp.exp(s - m)
        l = jnp.sum(p, axis=1, keepdims=True)
        o_ref[b, h] = lax.dot_general(
            p, vbuf[slot], (((1,), (0,)), ((), ())),
            preferred_element_type=jnp.float32,
        )
        c = _stat_col(b, h)
        st_ref[:, c:c + 1] = m
        st_ref[:, c + 1:c + 2] = l


def _flash_partial(Q, K, V):
    skv = K.shape[1]
    return pl.pallas_call(
        _flash_partial_body,
        in_specs=[
            pl.BlockSpec(memory_space=pltpu.MemorySpace.VMEM),
            pl.BlockSpec(memory_space=pltpu.MemorySpace.HBM),
            pl.BlockSpec(memory_space=pltpu.MemorySpace.HBM),
        ],
        out_specs=[
            pl.BlockSpec(memory_space=pltpu.MemorySpace.VMEM),
            pl.BlockSpec(memory_space=pltpu.MemorySpace.VMEM),
        ],
        out_shape=[
            jax.ShapeDtypeStruct((B, H, SQ, D), jnp.float32),
            jax.ShapeDtypeStruct((SQ, 2 * BH), jnp.float32),
        ],
        scratch_shapes=[
            pltpu.VMEM((NSLOTS, skv, D), jnp.float32),
            pltpu.VMEM((NSLOTS, skv, D), jnp.float32),
            pltpu.SemaphoreType.DMA((NSLOTS,)),
            pltpu.SemaphoreType.DMA((NSLOTS,)),
        ],
    )(Q, K, V)


def _allreduce_body(o_ref, st_ref, out_ref, o_comm, s_comm,
                    so1, ro1, ss1, rs1, so2, ro2, ss2, rs2):
    my = lax.axis_index("i")
    left = lax.rem(my + N_DEV - 1, N_DEV)
    right = lax.rem(my + 1, N_DEV)

    o_comm[0] = o_ref[...]
    s_comm[0] = st_ref[...]

    barrier_sem = pltpu.get_barrier_semaphore()
    for nbr in (left, right):
        pl.semaphore_signal(
            barrier_sem, inc=1,
            device_id=(nbr,), device_id_type=pl.DeviceIdType.MESH,
        )
    pl.semaphore_wait(barrier_sem, 2)

    def rdma(src, dst, send_sem, recv_sem, dev):
        return pltpu.make_async_remote_copy(
            src_ref=src, dst_ref=dst, send_sem=send_sem, recv_sem=recv_sem,
            device_id=(dev,), device_id_type=pl.DeviceIdType.MESH,
        )

    p1 = [
        rdma(o_comm.at[0], o_comm.at[1], so1.at[0], ro1.at[0], right),
        rdma(o_comm.at[0], o_comm.at[2], so1.at[1], ro1.at[1], left),
        rdma(s_comm.at[0], s_comm.at[1], ss1.at[0], rs1.at[0], right),
        rdma(s_comm.at[0], s_comm.at[2], ss1.at[1], rs1.at[1], left),
    ]
    for r in p1:
        r.start()
    for r in p1:
        r.wait()

    p2 = [
        rdma(o_comm.at[1, :, 0:H2], o_comm.at[3, :, 0:H2],
             so2.at[0], ro2.at[0], right),
        rdma(o_comm.at[2, :, H2:H], o_comm.at[3, :, H2:H],
             so2.at[1], ro2.at[1], left),
        rdma(s_comm.at[1], s_comm.at[3], ss2.at[0], rs2.at[0], right),
        rdma(s_comm.at[2], s_comm.at[3], ss2.at[1], rs2.at[1], left),
    ]
    for r in p2:
        r.start()
    for r in p2:
        r.wait()

    for b in range(B):
        for hh in range(H):
            c = _stat_col(b, hh)
            ms = [s_comm[j, :, c:c + 1] for j in range(N_DEV)]
            m_tot = ms[0]
            for j in range(1, N_DEV):
                m_tot = jnp.maximum(m_tot, ms[j])
            l_tot = jnp.zeros((SQ, 1), jnp.float32)
            o_tot = jnp.zeros((SQ, D), jnp.float32)
            for j in range(N_DEV):
                w = jnp.exp(ms[j] - m_tot)
                l_tot = l_tot + w * s_comm[j, :, c + 1:c + 2]
                o_tot = o_tot + w * o_comm[j, b, hh]
            out_ref[b, :, hh, :] = o_tot / l_tot


def _allreduce_combine(o_part, stats):
    return pl.pallas_call(
        _allreduce_body,
        in_specs=[
            pl.BlockSpec(memory_space=pltpu.MemorySpace.VMEM),
            pl.BlockSpec(memory_space=pltpu.MemorySpace.VMEM),
        ],
        out_specs=pl.BlockSpec(memory_space=pltpu.MemorySpace.VMEM),
        out_shape=jax.ShapeDtypeStruct((B, SQ, H, D), jnp.float32),
        scratch_shapes=[
            pltpu.VMEM((N_DEV, B, H, SQ, D), jnp.float32),
            pltpu.VMEM((N_DEV, SQ, 2 * BH), jnp.float32),
            pltpu.SemaphoreType.DMA((2,)),
            pltpu.SemaphoreType.DMA((2,)),
            pltpu.SemaphoreType.DMA((2,)),
            pltpu.SemaphoreType.DMA((2,)),
            pltpu.SemaphoreType.DMA((2,)),
            pltpu.SemaphoreType.DMA((2,)),
            pltpu.SemaphoreType.DMA((2,)),
            pltpu.SemaphoreType.DMA((2,)),
        ],
        compiler_params=pltpu.CompilerParams(collective_id=0),
    )(o_part, stats)


def _fused_body(q_ref, k_hbm, v_hbm, out_ref,
                kbuf, vbuf, ksems, vsems, o_comm,
                so1, ro1, so2, ro2):
    my = lax.axis_index("i")
    left = lax.rem(my + N_DEV - 1, N_DEV)
    right = lax.rem(my + 1, N_DEV)

    barrier_sem = pltpu.get_barrier_semaphore()
    for nbr in (left, right):
        pl.semaphore_signal(
            barrier_sem, inc=1,
            device_id=(nbr,), device_id_type=pl.DeviceIdType.MESH,
        )
    pl.semaphore_wait(barrier_sem, 2)

    def rdma(src, dst, send_sem, recv_sem, dev):
        return pltpu.make_async_remote_copy(
            src_ref=src, dst_ref=dst, send_sem=send_sem, recv_sem=recv_sem,
            device_id=(dev,), device_id_type=pl.DeviceIdType.MESH,
        )

    p1r = [rdma(o_comm.at[0, b], o_comm.at[1, b], so1.at[0, b], ro1.at[0, b],
                right) for b in range(B)]
    p1l = [rdma(o_comm.at[0, b], o_comm.at[2, b], so1.at[1, b], ro1.at[1, b],
                left) for b in range(B)]
    p2r = [rdma(o_comm.at[1, b, 0:H2], o_comm.at[3, b, 0:H2],
                so2.at[0, b], ro2.at[0, b], right) for b in range(B)]
    p2l = [rdma(o_comm.at[2, b, H2:H + 1], o_comm.at[3, b, H2:H + 1],
                so2.at[1, b], ro2.at[1, b], left) for b in range(B)]

    def start_fetch(idx, slot):
        b, h = idx // H, idx % H
        pltpu.make_async_copy(
            k_hbm.at[b, :, h, :], kbuf.at[slot], ksems.at[slot]
        ).start()
        pltpu.make_async_copy(
            v_hbm.at[b, :, h, :], vbuf.at[slot], vsems.at[slot]
        ).start()

    def wait_fetch(idx, slot):
        b, h = idx // H, idx % H
        pltpu.make_async_copy(
            k_hbm.at[b, :, h, :], kbuf.at[slot], ksems.at[slot]
        ).wait()
        pltpu.make_async_copy(
            v_hbm.at[b, :, h, :], vbuf.at[slot], vsems.at[slot]
        ).wait()

    def combine_chunk(b):
        for hh in range(H):
            ms = [o_comm[j, b, H, :, 2 * hh:2 * hh + 1] for j in range(N_DEV)]
            m_tot = ms[0]
            for j in range(1, N_DEV):
                m_tot = jnp.maximum(m_tot, ms[j])
            l_tot = jnp.zeros((SQ, 1), jnp.float32)
            o_tot = jnp.zeros((SQ, D), jnp.float32)
            for j in range(N_DEV):
                w = jnp.exp(ms[j] - m_tot)
                l_tot = l_tot + w * o_comm[j, b, H, :, 2 * hh + 1:2 * hh + 2]
                o_tot = o_tot + w * o_comm[j, b, hh]
            out_ref[b, :, hh, :] = o_tot / l_tot

    for j in range(NSLOTS - 1):
        start_fetch(j, j)
    for idx in range(BH):
        b, h = idx // H, idx % H
        slot = idx % NSLOTS
        nxt = idx + NSLOTS - 1
        if nxt < BH:
            start_fetch(nxt, nxt % NSLOTS)
        wait_fetch(idx, slot)

        q = q_ref[b, :, h, :] * SCALE
        s = lax.dot_general(
            q, kbuf[slot], (((1,), (1,)), ((), ())),
            preferred_element_type=jnp.float32,
        )
        m = jnp.max(s, axis=1, keepdims=True)
        p = jnp.exp(s - m)
        l = jnp.sum(p, axis=1, keepdims=True)
        o_comm[0, b, h] = lax.dot_general(
            p, vbuf[slot], (((1,), (0,)), ((), ())),
            preferred_element_type=jnp.float32,
        )
        o_comm[0, b, H, :, 2 * h:2 * h + 1] = m
        o_comm[0, b, H, :, 2 * h + 1:2 * h + 2] = l

        if h == H - 1:
            p1r[b].start()
            p1l[b].start()
            if b >= 1:
                p1r[b - 1].wait_recv()
                p1l[b - 1].wait_recv()
                p2r[b - 1].start()
                p2l[b - 1].start()
            if b >= 2:
                p2r[b - 2].wait_recv()
                p2l[b - 2].wait_recv()
                combine_chunk(b - 2)

    p1r[B - 1].wait_recv()
    p1l[B - 1].wait_recv()
    p2r[B - 1].start()
    p2l[B - 1].start()
    p2r[B - 2].wait_recv()
    p2l[B - 2].wait_recv()
    combine_chunk(B - 2)
    p2r[B - 1].wait_recv()
    p2l[B - 1].wait_recv()
    combine_chunk(B - 1)

    for b in range(B):
        p1r[b].wait_send()
        p1l[b].wait_send()
        p2r[b].wait_send()
        p2l[b].wait_send()


def _fused(Q, K, V):
    skv = K.shape[1]
    return pl.pallas_call(
        _fused_body,
        in_specs=[
            pl.BlockSpec(memory_space=pltpu.MemorySpace.VMEM),
            pl.BlockSpec(memory_space=pltpu.MemorySpace.HBM),
            pl.BlockSpec(memory_space=pltpu.MemorySpace.HBM),
        ],
        out_specs=pl.BlockSpec(memory_space=pltpu.MemorySpace.VMEM),
        out_shape=jax.ShapeDtypeStruct((B, SQ, H, D), jnp.float32),
        scratch_shapes=[
            pltpu.VMEM((NSLOTS, skv, D), jnp.float32),
            pltpu.VMEM((NSLOTS, skv, D), jnp.float32),
            pltpu.SemaphoreType.DMA((NSLOTS,)),
            pltpu.SemaphoreType.DMA((NSLOTS,)),
            pltpu.VMEM((N_DEV, B, H + 1, SQ, D), jnp.float32),
            pltpu.SemaphoreType.DMA((2, B)),
            pltpu.SemaphoreType.DMA((2, B)),
            pltpu.SemaphoreType.DMA((2, B)),
            pltpu.SemaphoreType.DMA((2, B)),
        ],
        compiler_params=pltpu.CompilerParams(collective_id=0),
    )(Q, K, V)


def kernel(Q, K, V):
    return _fused(Q, K, V)


# device time: 54628 ns/iter; 1.0384x vs baseline; 1.0384x over previous
import jax
import jax.numpy as jnp
from jax import lax
from jax.experimental import pallas as pl
from jax.experimental.pallas import tpu as pltpu

N_DEV = 4
B, SQ, H, D = 4, 32, 8, 128
BH = B * H
H2 = H // 2
SCALE = D ** -0.5
NSLOTS = 4


def _stat_col(b, h):
    return 2 * (h * B + b)


def _flash_partial_body(q_ref, k_hbm, v_hbm, o_ref, st_ref,
                        kbuf, vbuf, ksems, vsems):
    def start_fetch(idx, slot):
        b, h = idx // H, idx % H
        pltpu.make_async_copy(
            k_hbm.at[b, :, h, :], kbuf.at[slot], ksems.at[slot]
        ).start()
        pltpu.make_async_copy(
            v_hbm.at[b, :, h, :], vbuf.at[slot], vsems.at[slot]
        ).start()

    def wait_fetch(idx, slot):
        b, h = idx // H, idx % H
        pltpu.make_async_copy(
            k_hbm.at[b, :, h, :], kbuf.at[slot], ksems.at[slot]
        ).wait()
        pltpu.make_async_copy(
            v_hbm.at[b, :, h, :], vbuf.at[slot], vsems.at[slot]
        ).wait()

    for j in range(NSLOTS - 1):
        start_fetch(j, j)
    for idx in range(BH):
        b, h = idx // H, idx % H
        slot = idx % NSLOTS
        nxt = idx + NSLOTS - 1
        if nxt < BH:
            start_fetch(nxt, nxt % NSLOTS)
        wait_fetch(idx, slot)

        q = q_ref[b, :, h, :] * SCALE
        s = lax.dot_general(
            q, kbuf[slot], (((1,), (1,)), ((), ())),
            preferred_element_type=jnp.float32,
        )
        m = jnp.max(s, axis=1, keepdims=True)
        p = jnp.exp(s - m)
        l = jnp.sum(p, axis=1, keepdims=True)
        o_ref[b, h] = lax.dot_general(
            p, vbuf[slot], (((1,), (0,)), ((), ())),
            preferred_element_type=jnp.float32,
        )
        c = _stat_col(b, h)
        st_ref[:, c:c + 1] = m
        st_ref[:, c + 1:c + 2] = l


def _flash_partial(Q, K, V):
    skv = K.shape[1]
    return pl.pallas_call(
        _flash_partial_body,
        in_specs=[
            pl.BlockSpec(memory_space=pltpu.MemorySpace.VMEM),
            pl.BlockSpec(memory_space=pltpu.MemorySpace.HBM),
            pl.BlockSpec(memory_space=pltpu.MemorySpace.HBM),
        ],
        out_specs=[
            pl.BlockSpec(memory_space=pltpu.MemorySpace.VMEM),
            pl.BlockSpec(memory_space=pltpu.MemorySpace.VMEM),
        ],
        out_shape=[
            jax.ShapeDtypeStruct((B, H, SQ, D), jnp.float32),
            jax.ShapeDtypeStruct((SQ, 2 * BH), jnp.float32),
        ],
        scratch_shapes=[
            pltpu.VMEM((NSLOTS, skv, D), jnp.float32),
            pltpu.VMEM((NSLOTS, skv, D), jnp.float32),
            pltpu.SemaphoreType.DMA((NSLOTS,)),
            pltpu.SemaphoreType.DMA((NSLOTS,)),
        ],
    )(Q, K, V)


def _allreduce_body(o_ref, st_ref, out_ref, o_comm, s_comm,
                    so1, ro1, ss1, rs1, so2, ro2, ss2, rs2):
    my = lax.axis_index("i")
    left = lax.rem(my + N_DEV - 1, N_DEV)
    right = lax.rem(my + 1, N_DEV)

    o_comm[0] = o_ref[...]
    s_comm[0] = st_ref[...]

    barrier_sem = pltpu.get_barrier_semaphore()
    for nbr in (left, right):
        pl.semaphore_signal(
            barrier_sem, inc=1,
            device_id=(nbr,), device_id_type=pl.DeviceIdType.MESH,
        )
    pl.semaphore_wait(barrier_sem, 2)

    def rdma(src, dst, send_sem, recv_sem, dev):
        return pltpu.make_async_remote_copy(
            src_ref=src, dst_ref=dst, send_sem=send_sem, recv_sem=recv_sem,
            device_id=(dev,), device_id_type=pl.DeviceIdType.MESH,
        )

    p1 = [
        rdma(o_comm.at[0], o_comm.at[1], so1.at[0], ro1.at[0], right),
        rdma(o_comm.at[0], o_comm.at[2], so1.at[1], ro1.at[1], left),
        rdma(s_comm.at[0], s_comm.at[1], ss1.at[0], rs1.at[0], right),
        rdma(s_comm.at[0], s_comm.at[2], ss1.at[1], rs1.at[1], left),
    ]
    for r in p1:
        r.start()
    for r in p1:
        r.wait()

    p2 = [
        rdma(o_comm.at[1, :, 0:H2], o_comm.at[3, :, 0:H2],
             so2.at[0], ro2.at[0], right),
        rdma(o_comm.at[2, :, H2:H], o_comm.at[3, :, H2:H],
             so2.at[1], ro2.at[1], left),
        rdma(s_comm.at[1], s_comm.at[3], ss2.at[0], rs2.at[0], right),
        rdma(s_comm.at[2], s_comm.at[3], ss2.at[1], rs2.at[1], left),
    ]
    for r in p2:
        r.start()
    for r in p2:
        r.wait()

    for b in range(B):
        for hh in range(H):
            c = _stat_col(b, hh)
            ms = [s_comm[j, :, c:c + 1] for j in range(N_DEV)]
            m_tot = ms[0]
            for j in range(1, N_DEV):
                m_tot = jnp.maximum(m_tot, ms[j])
            l_tot = jnp.zeros((SQ, 1), jnp.float32)
            o_tot = jnp.zeros((SQ, D), jnp.float32)
            for j in range(N_DEV):
                w = jnp.exp(ms[j] - m_tot)
                l_tot = l_tot + w * s_comm[j, :, c + 1:c + 2]
                o_tot = o_tot + w * o_comm[j, b, hh]
            out_ref[b, :, hh, :] = o_tot / l_tot


def _allreduce_combine(o_part, stats):
    return pl.pallas_call(
        _allreduce_body,
        in_specs=[
            pl.BlockSpec(memory_space=pltpu.MemorySpace.VMEM),
            pl.BlockSpec(memory_space=pltpu.MemorySpace.VMEM),
        ],
        out_specs=pl.BlockSpec(memory_space=pltpu.MemorySpace.VMEM),
        out_shape=jax.ShapeDtypeStruct((B, SQ, H, D), jnp.float32),
        scratch_shapes=[
            pltpu.VMEM((N_DEV, B, H, SQ, D), jnp.float32),
            pltpu.VMEM((N_DEV, SQ, 2 * BH), jnp.float32),
            pltpu.SemaphoreType.DMA((2,)),
            pltpu.SemaphoreType.DMA((2,)),
            pltpu.SemaphoreType.DMA((2,)),
            pltpu.SemaphoreType.DMA((2,)),
            pltpu.SemaphoreType.DMA((2,)),
            pltpu.SemaphoreType.DMA((2,)),
            pltpu.SemaphoreType.DMA((2,)),
            pltpu.SemaphoreType.DMA((2,)),
        ],
        compiler_params=pltpu.CompilerParams(collective_id=0),
    )(o_part, stats)


def _fused_body(q_ref, k_hbm, v_hbm, out_ref,
                kbuf, vbuf, ksems, vsems, o_comm,
                so1, ro1, so2, ro2, so1h, ro1h):
    my = lax.axis_index("i")
    left = lax.rem(my + N_DEV - 1, N_DEV)
    right = lax.rem(my + 1, N_DEV)

    barrier_sem = pltpu.get_barrier_semaphore()
    for nbr in (left, right):
        pl.semaphore_signal(
            barrier_sem, inc=1,
            device_id=(nbr,), device_id_type=pl.DeviceIdType.MESH,
        )
    pl.semaphore_wait(barrier_sem, 2)

    def rdma(src, dst, send_sem, recv_sem, dev):
        return pltpu.make_async_remote_copy(
            src_ref=src, dst_ref=dst, send_sem=send_sem, recv_sem=recv_sem,
            device_id=(dev,), device_id_type=pl.DeviceIdType.MESH,
        )

    LB = B - 1
    p1r = [rdma(o_comm.at[0, b], o_comm.at[1, b], so1.at[0, b], ro1.at[0, b],
                right) for b in range(B - 1)]
    p1l = [rdma(o_comm.at[0, b], o_comm.at[2, b], so1.at[1, b], ro1.at[1, b],
                left) for b in range(B - 1)]
    p1r_a = rdma(o_comm.at[0, LB, 0:H2], o_comm.at[1, LB, 0:H2],
                 so1h.at[0, 0], ro1h.at[0, 0], right)
    p1l_a = rdma(o_comm.at[0, LB, 0:H2], o_comm.at[2, LB, 0:H2],
                 so1h.at[1, 0], ro1h.at[1, 0], left)
    p1r_b = rdma(o_comm.at[0, LB, H2:H + 1], o_comm.at[1, LB, H2:H + 1],
                 so1h.at[0, 1], ro1h.at[0, 1], right)
    p1l_b = rdma(o_comm.at[0, LB, H2:H + 1], o_comm.at[2, LB, H2:H + 1],
                 so1h.at[1, 1], ro1h.at[1, 1], left)
    p2r = [rdma(o_comm.at[1, b, 0:H2], o_comm.at[3, b, 0:H2],
                so2.at[0, b], ro2.at[0, b], right) for b in range(B)]
    p2l = [rdma(o_comm.at[2, b, H2:H + 1], o_comm.at[3, b, H2:H + 1],
                so2.at[1, b], ro2.at[1, b], left) for b in range(B)]

    def start_fetch(idx, slot):
        b, h = idx // H, idx % H
        pltpu.make_async_copy(
            k_hbm.at[b, :, h, :], kbuf.at[slot], ksems.at[slot]
        ).start()
        pltpu.make_async_copy(
            v_hbm.at[b, :, h, :], vbuf.at[slot], vsems.at[slot]
        ).start()

    def wait_fetch(idx, slot):
        b, h = idx // H, idx % H
        pltpu.make_async_copy(
            k_hbm.at[b, :, h, :], kbuf.at[slot], ksems.at[slot]
        ).wait()
        pltpu.make_async_copy(
            v_hbm.at[b, :, h, :], vbuf.at[slot], vsems.at[slot]
        ).wait()

    def combine_chunk(b):
        for hh in range(H):
            ms = [o_comm[j, b, H, :, 2 * hh:2 * hh + 1] for j in range(N_DEV)]
            m_tot = ms[0]
            for j in range(1, N_DEV):
                m_tot = jnp.maximum(m_tot, ms[j])
            l_tot = jnp.zeros((SQ, 1), jnp.float32)
            o_tot = jnp.zeros((SQ, D), jnp.float32)
            for j in range(N_DEV):
                w = jnp.exp(ms[j] - m_tot)
                l_tot = l_tot + w * o_comm[j, b, H, :, 2 * hh + 1:2 * hh + 2]
                o_tot = o_tot + w * o_comm[j, b, hh]
            out_ref[b, :, hh, :] = o_tot / l_tot

    for j in range(NSLOTS - 1):
        start_fetch(j, j)
    for idx in range(BH):
        b, h = idx // H, idx % H
        slot = idx % NSLOTS
        nxt = idx + NSLOTS - 1
        if nxt < BH:
            start_fetch(nxt, nxt % NSLOTS)
        wait_fetch(idx, slot)

        q = q_ref[b, :, h, :] * SCALE
        s = lax.dot_general(
            q, kbuf[slot], (((1,), (1,)), ((), ())),
            preferred_element_type=jnp.float32,
        )
        m = jnp.max(s, axis=1, keepdims=True)
        p = jnp.exp(s - m)
        l = jnp.sum(p, axis=1, keepdims=True)
        o_comm[0, b, h] = lax.dot_general(
            p, vbuf[slot], (((1,), (0,)), ((), ())),
            preferred_element_type=jnp.float32,
        )
        o_comm[0, b, H, :, 2 * h:2 * h + 1] = m
        o_comm[0, b, H, :, 2 * h + 1:2 * h + 2] = l

        if b == LB and h == H2 - 1:
            p1r_a.start()
            p1l_a.start()
        if h == H - 1:
            if b < LB:
                p1r[b].start()
                p1l[b].start()
            else:
                p1r_b.start()
                p1l_b.start()
            if b >= 1:
                p1r[b - 1].wait_recv()
                p1l[b - 1].wait_recv()
                p2r[b - 1].start()
                p2l[b - 1].start()
            if b >= 2:
                p2r[b - 2].wait_recv()
                p2l[b - 2].wait_recv()
                combine_chunk(b - 2)

    p1r_a.wait_recv()
    p2r[LB].start()
    p1l_b.wait_recv()
    p2l[LB].start()
    p2r[LB - 1].wait_recv()
    p2l[LB - 1].wait_recv()
    combine_chunk(LB - 1)
    p1r_b.wait_recv()
    p1l_a.wait_recv()
    p2r[LB].wait_recv()
    p2l[LB].wait_recv()
    combine_chunk(LB)

    for b in range(B - 1):
        p1r[b].wait_send()
        p1l[b].wait_send()
    for r in (p1r_a, p1l_a, p1r_b, p1l_b):
        r.wait_send()
    for b in range(B):
        p2r[b].wait_send()
        p2l[b].wait_send()


def _fused(Q, K, V):
    skv = K.shape[1]
    return pl.pallas_call(
        _fused_body,
        in_specs=[
            pl.BlockSpec(memory_space=pltpu.MemorySpace.VMEM),
            pl.BlockSpec(memory_space=pltpu.MemorySpace.HBM),
            pl.BlockSpec(memory_space=pltpu.MemorySpace.HBM),
        ],
        out_specs=pl.BlockSpec(memory_space=pltpu.MemorySpace.VMEM),
        out_shape=jax.ShapeDtypeStruct((B, SQ, H, D), jnp.float32),
        scratch_shapes=[
            pltpu.VMEM((NSLOTS, skv, D), jnp.float32),
            pltpu.VMEM((NSLOTS, skv, D), jnp.float32),
            pltpu.SemaphoreType.DMA((NSLOTS,)),
            pltpu.SemaphoreType.DMA((NSLOTS,)),
            pltpu.VMEM((N_DEV, B, H + 1, SQ, D), jnp.float32),
            pltpu.SemaphoreType.DMA((2, B)),
            pltpu.SemaphoreType.DMA((2, B)),
            pltpu.SemaphoreType.DMA((2, B)),
            pltpu.SemaphoreType.DMA((2, B)),
            pltpu.SemaphoreType.DMA((2, 2)),
            pltpu.SemaphoreType.DMA((2, 2)),
        ],
        compiler_params=pltpu.CompilerParams(collective_id=0),
    )(Q, K, V)


def kernel(Q, K, V):
    return _fused(Q, K, V)
